# Initial kernel scaffold; baseline (speedup 1.0000x reference)
import jax
import jax.numpy as jnp
from jax.experimental import pallas as pl


def kernel(boxes, scores):
    raise NotImplementedError("write your pallas kernel here")


# speedup vs baseline: 15.6605x; 15.6605x over previous
import jax
import jax.numpy as jnp
from jax.experimental import pallas as pl
from jax.experimental.pallas import tpu as pltpu

_N = 20000
_NPAD = 20480
_ROWS = _NPAD // 128
_K = 6000
_OUT = 1000
_IOU_T = 0.7


def _nms_body(cols_ref, s_ref, boxr_ref, out_ref, sw_ref, a2_ref, iota_ref):
    s = s_ref[...]
    sm = jnp.where(s > 0.0, s, -jnp.inf)
    sbits = jax.lax.bitcast_convert_type(sm, jnp.int32)

    ir = jax.lax.broadcasted_iota(jnp.int32, (_ROWS, 128), 0)
    ic = jax.lax.broadcasted_iota(jnp.int32, (_ROWS, 128), 1)
    iota = ir * 128 + ic
    iota_ref[...] = iota

    def bs1(_, c):
        lo, hi = c
        mid = lo + (hi - lo) // 2
        cnt = jnp.sum(jnp.where(sbits >= mid, 1.0, 0.0))
        ge = cnt >= float(_K)
        return (jnp.where(ge, mid, lo), jnp.where(ge, hi, mid))

    lo, _ = jax.lax.fori_loop(0, 31, bs1, (jnp.int32(0), jnp.int32(0x7F800000)))

    cnt_gt = jnp.sum(jnp.where(sbits > lo, 1.0, 0.0))
    need = float(_K) - cnt_gt
    eq = sbits == lo

    def bs2(_, c):
        l2, h2 = c
        mid = l2 + (h2 - l2) // 2
        cc = jnp.sum(jnp.where(eq & (iota <= mid), 1.0, 0.0))
        ge = cc >= need
        return (jnp.where(ge, l2, mid), jnp.where(ge, mid, h2))

    _, tie_hi = jax.lax.fori_loop(
        0, 15, bs2, (jnp.int32(-1), jnp.int32(_NPAD - 1)))

    keep = (sbits > lo) | (eq & (iota <= tie_hi))
    sw0 = jnp.where(keep, sm, -jnp.inf)
    sw_ref[...] = sw0

    x1 = cols_ref[0]
    y1 = cols_ref[1]
    x2 = cols_ref[2]
    y2 = cols_ref[3]
    a2_ref[...] = (x2 - x1) * (y2 - y1)

    m0 = jnp.max(sw0, axis=(0, 1), keepdims=True)

    def step(i, m11):
        sw = sw_ref[...]
        tgt = jnp.where(sw == m11, iota_ref[...], jnp.int32(0x3FFFFFFF))
        idx11 = jnp.min(tgt, axis=(0, 1), keepdims=True)
        idx = idx11[0, 0]
        bbox = boxr_ref[pl.ds(idx, 1), :]
        bx1 = bbox[:, 0:1]
        by1 = bbox[:, 1:2]
        bx2 = bbox[:, 2:3]
        by2 = bbox[:, 3:4]
        alive = m11 > -jnp.inf
        w = jnp.where(alive, 1.0, 0.0)
        area1 = (bx2 - bx1) * (by2 - by1)
        ltx = jnp.maximum(bx1, cols_ref[0])
        lty = jnp.maximum(by1, cols_ref[1])
        rbx = jnp.minimum(bx2, cols_ref[2])
        rby = jnp.minimum(by2, cols_ref[3])
        iw = jnp.maximum(rbx - ltx, 0.0)
        ih = jnp.maximum(rby - lty, 0.0)
        inter = iw * ih
        iou = inter / (area1 + a2_ref[...] - inter + 1e-9)
        sw2 = jnp.where(iou > _IOU_T, -jnp.inf, sw)
        sw_ref[...] = sw2
        m_next = jnp.max(sw2, axis=(0, 1), keepdims=True)
        sc = jnp.where(alive, m11, 0.0)
        li = jax.lax.broadcasted_iota(jnp.int32, (1, 8), 1)
        row = (jnp.where(li == 0, bx1 * w, 0.0)
               + jnp.where(li == 1, by1 * w, 0.0)
               + jnp.where(li == 2, bx2 * w, 0.0)
               + jnp.where(li == 3, by2 * w, 0.0)
               + jnp.where(li == 4, sc, 0.0))
        out_ref[pl.ds(i, 1), :] = row
        return m_next

    jax.lax.fori_loop(0, _OUT, step, m0)


def kernel(boxes, scores):
    boxes_r = jnp.pad(boxes, ((0, _NPAD - _N), (0, 0)))
    cols = boxes_r.T.reshape(4, _ROWS, 128)
    s2d = jnp.pad(scores, (0, _NPAD - _N)).reshape(_ROWS, 128)
    out = pl.pallas_call(
        _nms_body,
        out_shape=jax.ShapeDtypeStruct((1024, 8), jnp.float32),
        scratch_shapes=[
            pltpu.VMEM((_ROWS, 128), jnp.float32),
            pltpu.VMEM((_ROWS, 128), jnp.float32),
            pltpu.VMEM((_ROWS, 128), jnp.int32),
        ],
    )(cols, s2d, boxes_r)
    return out[:_OUT, :5]
